# baseline (speedup 1.0000x reference)
import functools

import jax
import jax.numpy as jnp
from jax.experimental import pallas as pl
from jax.experimental.pallas import tpu as pltpu

_K = 16
_DILATION = 2
_KK = _K * _DILATION
_INF = float("inf")


def _ce(av, ai, bv, bi):
    lt = (av < bv) | ((av == bv) & (ai < bi))
    lov = jnp.where(lt, av, bv)
    loi = jnp.where(lt, ai, bi)
    hiv = jnp.where(lt, bv, av)
    hii = jnp.where(lt, bi, ai)
    return lov, loi, hiv, hii


def _knn_body(xs_ref, ys_ref, xsq_ref, ysq_ref, out_ref, *, n_keys):
    xs = xs_ref[0]
    ys = ys_ref[0]
    xsq = xsq_ref[0]
    ysq = ysq_ref[0]

    inner = jax.lax.dot_general(
        xs, ys,
        dimension_numbers=(((1,), (1,)), ((), ())),
        precision=jax.lax.Precision.DEFAULT,
        preferred_element_type=jnp.float32,
    )
    dist = (xsq + (-2.0 * inner)) + ysq

    tq = dist.shape[0]
    q = n_keys // 4
    iota = jax.lax.broadcasted_iota(jnp.int32, (tq, q), 1).astype(jnp.float32)

    x0v, x1v = dist[:, :q], dist[:, q:2 * q]
    x2v, x3v = dist[:, 2 * q:3 * q], dist[:, 3 * q:]
    x0i = iota
    x1i = iota + float(q)
    x2i = iota + float(2 * q)
    x3i = iota + float(3 * q)

    x0v, x0i, x1v, x1i = _ce(x0v, x0i, x1v, x1i)
    x2v, x2i, x3v, x3i = _ce(x2v, x2i, x3v, x3i)
    x0v, x0i, x2v, x2i = _ce(x0v, x0i, x2v, x2i)
    x1v, x1i, x3v, x3i = _ce(x1v, x1i, x3v, x3i)
    x1v, x1i, x2v, x2i = _ce(x1v, x1i, x2v, x2i)

    picked = []
    for t in range(_KK - 1):
        m = jnp.min(x0v, axis=1, keepdims=True)
        cand = jnp.where(x0v == m, x0i, float(2 * n_keys))
        j = jnp.min(cand, axis=1, keepdims=True)
        picked.append(j)
        mask = x0i == j
        x0v = jnp.where(mask, x1v, x0v)
        x0i = jnp.where(mask, x1i, x0i)
        x1v = jnp.where(mask, x2v, x1v)
        x1i = jnp.where(mask, x2i, x1i)
        x2v = jnp.where(mask, x3v, x2v)
        x2i = jnp.where(mask, x3i, x2i)
        x3v = jnp.where(mask, _INF, x3v)

    out_ref[0] = jnp.concatenate(picked[::_DILATION], axis=1).astype(jnp.int32)


def kernel(x, y):
    b, d, n, _ = x.shape

    def _norm(t):
        nrm = jnp.sqrt(jnp.sum(t * t, axis=1, keepdims=True))
        return t / jnp.maximum(nrm, 1e-12)

    xn = _norm(x)
    yn = _norm(y)
    xs = jnp.squeeze(jnp.transpose(xn, (0, 2, 1, 3)), -1)
    ys = jnp.squeeze(jnp.transpose(yn, (0, 2, 1, 3)), -1)
    x_square = jnp.sum(xs * xs, axis=-1, keepdims=True)
    y_square = jnp.sum(ys * ys, axis=-1, keepdims=True)
    ysq_t = jnp.swapaxes(y_square, 2, 1)

    tq = 64
    grid = (b, n // tq)

    nn_idx = pl.pallas_call(
        functools.partial(_knn_body, n_keys=n),
        grid=grid,
        in_specs=[
            pl.BlockSpec((1, tq, d), lambda bi, qi: (bi, qi, 0)),
            pl.BlockSpec((1, n, d), lambda bi, qi: (bi, 0, 0)),
            pl.BlockSpec((1, tq, 1), lambda bi, qi: (bi, qi, 0)),
            pl.BlockSpec((1, 1, n), lambda bi, qi: (bi, 0, 0)),
        ],
        out_specs=pl.BlockSpec((1, tq, _K), lambda bi, qi: (bi, qi, 0)),
        out_shape=jax.ShapeDtypeStruct((b, n, _K), jnp.int32),
        compiler_params=pltpu.CompilerParams(
            dimension_semantics=("parallel", "parallel"),
        ),
    )(xs, ys, x_square, ysq_t)

    center_idx = jnp.broadcast_to(
        jnp.arange(n, dtype=jnp.int32)[None, :, None], (b, n, _K)
    )
    return jnp.stack((nn_idx, center_idx), axis=0)


# speedup vs baseline: 1.1410x; 1.1410x over previous
import functools

import jax
import jax.numpy as jnp
from jax.experimental import pallas as pl
from jax.experimental.pallas import tpu as pltpu

_K = 16
_DILATION = 2
_KK = _K * _DILATION
_INF = float("inf")


def _ce(av, ai, bv, bi):
    lt = (av < bv) | ((av == bv) & (ai < bi))
    lov = jnp.where(lt, av, bv)
    loi = jnp.where(lt, ai, bi)
    hiv = jnp.where(lt, bv, av)
    hii = jnp.where(lt, bi, ai)
    return lov, loi, hiv, hii


def _knn_body(xs_ref, ys_ref, xsq_ref, ysq_ref, out_ref, *, n_keys):
    xs = xs_ref[0]
    ys = ys_ref[0]
    xsq = xsq_ref[0]
    ysq = ysq_ref[0]

    inner = jax.lax.dot_general(
        xs, ys,
        dimension_numbers=(((1,), (1,)), ((), ())),
        precision=jax.lax.Precision.DEFAULT,
        preferred_element_type=jnp.float32,
    )
    dist = (xsq + (-2.0 * inner)) + ysq

    tq = dist.shape[0]
    g = 8
    q = n_keys // g
    iota = jax.lax.broadcasted_iota(jnp.int32, (tq, q), 1).astype(jnp.float32)

    vs = [dist[:, k * q:(k + 1) * q] for k in range(g)]
    ix = [iota + float(k * q) for k in range(g)]
    network = [(0, 1), (2, 3), (4, 5), (6, 7),
               (0, 2), (1, 3), (4, 6), (5, 7),
               (1, 2), (5, 6),
               (0, 4), (1, 5), (2, 6), (3, 7),
               (2, 4), (3, 5),
               (1, 2), (3, 4), (5, 6)]
    for a, b in network:
        vs[a], ix[a], vs[b], ix[b] = _ce(vs[a], ix[a], vs[b], ix[b])

    picked = []
    for t in range(_KK - 1):
        m = jnp.min(vs[0], axis=1, keepdims=True)
        cand = jnp.where(vs[0] == m, ix[0], float(2 * n_keys))
        j = jnp.min(cand, axis=1, keepdims=True)
        picked.append(j)
        if t == _KK - 2:
            break
        mask = ix[0] == j
        for k in range(g - 1):
            vs[k] = jnp.where(mask, vs[k + 1], vs[k])
            ix[k] = jnp.where(mask, ix[k + 1], ix[k])
        vs[g - 1] = jnp.where(mask, _INF, vs[g - 1])

    out_ref[0] = jnp.concatenate(picked[::_DILATION], axis=1).astype(jnp.int32)


def kernel(x, y):
    b, d, n, _ = x.shape

    def _norm(t):
        nrm = jnp.sqrt(jnp.sum(t * t, axis=1, keepdims=True))
        return t / jnp.maximum(nrm, 1e-12)

    xn = _norm(x)
    yn = _norm(y)
    xs = jnp.squeeze(jnp.transpose(xn, (0, 2, 1, 3)), -1)
    ys = jnp.squeeze(jnp.transpose(yn, (0, 2, 1, 3)), -1)
    x_square = jnp.sum(xs * xs, axis=-1, keepdims=True)
    y_square = jnp.sum(ys * ys, axis=-1, keepdims=True)
    ysq_t = jnp.swapaxes(y_square, 2, 1)

    tq = 128
    grid = (b, n // tq)

    nn_idx = pl.pallas_call(
        functools.partial(_knn_body, n_keys=n),
        grid=grid,
        in_specs=[
            pl.BlockSpec((1, tq, d), lambda bi, qi: (bi, qi, 0)),
            pl.BlockSpec((1, n, d), lambda bi, qi: (bi, 0, 0)),
            pl.BlockSpec((1, tq, 1), lambda bi, qi: (bi, qi, 0)),
            pl.BlockSpec((1, 1, n), lambda bi, qi: (bi, 0, 0)),
        ],
        out_specs=pl.BlockSpec((1, tq, _K), lambda bi, qi: (bi, qi, 0)),
        out_shape=jax.ShapeDtypeStruct((b, n, _K), jnp.int32),
        compiler_params=pltpu.CompilerParams(
            dimension_semantics=("parallel", "parallel"),
        ),
    )(xs, ys, x_square, ysq_t)

    center_idx = jnp.broadcast_to(
        jnp.arange(n, dtype=jnp.int32)[None, :, None], (b, n, _K)
    )
    return jnp.stack((nn_idx, center_idx), axis=0)


# speedup vs baseline: 1.1920x; 1.0448x over previous
import functools

import jax
import jax.numpy as jnp
from jax.experimental import pallas as pl
from jax.experimental.pallas import tpu as pltpu

_K = 16
_DILATION = 2
_KK = _K * _DILATION
_INF = float("inf")


def _ce(av, ai, bv, bi):
    lt = (av < bv) | ((av == bv) & (ai < bi))
    lov = jnp.where(lt, av, bv)
    loi = jnp.where(lt, ai, bi)
    hiv = jnp.where(lt, bv, av)
    hii = jnp.where(lt, bi, ai)
    return lov, loi, hiv, hii


def _knn_body(xs_ref, ys_ref, xsq_ref, ysq_ref, out_ref, *, n_keys):
    xs = xs_ref[0]
    ys = ys_ref[0]
    xsq = xsq_ref[0]
    ysq = ysq_ref[0]

    inner = jax.lax.dot_general(
        xs, ys,
        dimension_numbers=(((1,), (1,)), ((), ())),
        precision=jax.lax.Precision.DEFAULT,
        preferred_element_type=jnp.float32,
    )
    dist = (xsq + (-2.0 * inner)) + ysq

    tq = dist.shape[0]
    g = 4
    q = n_keys // g
    iota = jax.lax.broadcasted_iota(jnp.int32, (tq, q), 1).astype(jnp.float32)

    vs = [dist[:, k * q:(k + 1) * q] for k in range(g)]
    ix = [iota + float(k * q) for k in range(g)]
    network = [(0, 1), (2, 3), (0, 2), (1, 3), (1, 2)]
    for a, b in network:
        vs[a], ix[a], vs[b], ix[b] = _ce(vs[a], ix[a], vs[b], ix[b])

    picked = []
    for t in range(_KK - 1):
        m = jnp.min(vs[0], axis=1, keepdims=True)
        cand = jnp.where(vs[0] == m, ix[0], float(2 * n_keys))
        j = jnp.min(cand, axis=1, keepdims=True)
        picked.append(j)
        if t == _KK - 2:
            break
        mask = ix[0] == j
        for k in range(g - 1):
            vs[k] = jnp.where(mask, vs[k + 1], vs[k])
            ix[k] = jnp.where(mask, ix[k + 1], ix[k])
        vs[g - 1] = jnp.where(mask, _INF, vs[g - 1])

    out_ref[0] = jnp.concatenate(picked[::_DILATION], axis=1).astype(jnp.int32)


def kernel(x, y):
    b, d, n, _ = x.shape

    def _norm(t):
        nrm = jnp.sqrt(jnp.sum(t * t, axis=1, keepdims=True))
        return t / jnp.maximum(nrm, 1e-12)

    xn = _norm(x)
    yn = _norm(y)
    xs = jnp.squeeze(jnp.transpose(xn, (0, 2, 1, 3)), -1)
    ys = jnp.squeeze(jnp.transpose(yn, (0, 2, 1, 3)), -1)
    x_square = jnp.sum(xs * xs, axis=-1, keepdims=True)
    y_square = jnp.sum(ys * ys, axis=-1, keepdims=True)
    ysq_t = jnp.swapaxes(y_square, 2, 1)

    tq = 128
    grid = (b, n // tq)

    nn_idx = pl.pallas_call(
        functools.partial(_knn_body, n_keys=n),
        grid=grid,
        in_specs=[
            pl.BlockSpec((1, tq, d), lambda bi, qi: (bi, qi, 0)),
            pl.BlockSpec((1, n, d), lambda bi, qi: (bi, 0, 0)),
            pl.BlockSpec((1, tq, 1), lambda bi, qi: (bi, qi, 0)),
            pl.BlockSpec((1, 1, n), lambda bi, qi: (bi, 0, 0)),
        ],
        out_specs=pl.BlockSpec((1, tq, _K), lambda bi, qi: (bi, qi, 0)),
        out_shape=jax.ShapeDtypeStruct((b, n, _K), jnp.int32),
        compiler_params=pltpu.CompilerParams(
            dimension_semantics=("parallel", "parallel"),
        ),
    )(xs, ys, x_square, ysq_t)

    center_idx = jnp.broadcast_to(
        jnp.arange(n, dtype=jnp.int32)[None, :, None], (b, n, _K)
    )
    return jnp.stack((nn_idx, center_idx), axis=0)
